# Initial kernel scaffold; baseline (speedup 1.0000x reference)
import jax
import jax.numpy as jnp
from jax.experimental import pallas as pl


def kernel(x, mesh_edge_index, mesh_edge_attr, world_edge_index, world_edge_attr, Wm1, bm1, Wm2, bm2, Ww1, bw1, Ww2, bw2):
    raise NotImplementedError("write your pallas kernel here")


# speedup vs baseline: 3.5725x; 3.5725x over previous
import functools

import jax
import jax.numpy as jnp
from jax import lax
from jax.experimental import pallas as pl
from jax.experimental.pallas import tpu as pltpu
from jax.experimental.pallas import tpu_sc as plsc

D = 128
NC, NS = 2, 16
NW = NC * NS

G_SUB = 80
G_FIRE = 5
G_CH = G_SUB * G_FIRE


def _prep_body(x_ref, wm1_ref, bm1_ref, ww1_ref, bw1_ref,
               pam_ref, pbm_ref, paw_ref, pbw_ref):
    x = x_ref[...]
    pam_ref[...] = jnp.dot(x, wm1_ref[0:D, :], preferred_element_type=jnp.float32) + bm1_ref[...]
    pbm_ref[...] = jnp.dot(x, wm1_ref[D:2 * D, :], preferred_element_type=jnp.float32)
    paw_ref[...] = jnp.dot(x, ww1_ref[0:D, :], preferred_element_type=jnp.float32) + bw1_ref[...]
    pbw_ref[...] = jnp.dot(x, ww1_ref[D:2 * D, :], preferred_element_type=jnp.float32)


def _precompute_tables(x, wm1, bm1, ww1, bw1):
    n = x.shape[0]
    blk = n // 5
    tbl = jax.ShapeDtypeStruct((n, D), jnp.float32)
    row_spec = pl.BlockSpec((blk, D), lambda i: (i, 0))
    full = pl.BlockSpec((2 * D, D), lambda i: (0, 0))
    bias = pl.BlockSpec((1, D), lambda i: (0, 0))
    return pl.pallas_call(
        _prep_body,
        grid=(5,),
        in_specs=[row_spec, full, bias, full, bias],
        out_specs=(row_spec, row_spec, row_spec, row_spec),
        out_shape=(tbl, tbl, tbl, tbl),
    )(x, wm1[: 2 * D], bm1.reshape(1, D), ww1[: 2 * D], bw1.reshape(1, D))


def _gather_stream(table, idx_hbm, out_hbm, idx_v, rows_v, sem, base, n_here):
    def outer(i, carry):
        off = base + i * G_CH
        pltpu.sync_copy(idx_hbm.at[pl.ds(off, G_CH)], idx_v)
        copies = [
            pltpu.async_copy(
                table.at[idx_v.at[pl.ds(j * G_SUB, G_SUB)]],
                rows_v.at[pl.ds(j * G_SUB, G_SUB)],
                sem,
            )
            for j in range(G_FIRE)
        ]
        for c in copies:
            c.wait()
        pltpu.sync_copy(rows_v, out_hbm.at[pl.ds(off, G_CH)])
        return carry
    lax.fori_loop(0, n_here // G_CH, outer, 0, unroll=False)


def _sc_gather_body(pam, pbm, paw, pbw, sm, rm, sw, rw,
                    gam, gbm, gaw, gbw, idx_v, rows_v, sem):
    wid = lax.axis_index("s") * NC + lax.axis_index("c")
    em = sm.shape[0] // NW
    ew = sw.shape[0] // NW
    _gather_stream(pam, sm, gam, idx_v, rows_v, sem, wid * em, em)
    _gather_stream(pbm, rm, gbm, idx_v, rows_v, sem, wid * em, em)
    _gather_stream(paw, sw, gaw, idx_v, rows_v, sem, wid * ew, ew)
    _gather_stream(pbw, rw, gbw, idx_v, rows_v, sem, wid * ew, ew)


def _sc_gather(pam, pbm, paw, pbw, sm, rm, sw, rw):
    em, ew = sm.shape[0], sw.shape[0]
    out = (jax.ShapeDtypeStruct((em, D), jnp.float32),
           jax.ShapeDtypeStruct((em, D), jnp.float32),
           jax.ShapeDtypeStruct((ew, D), jnp.float32),
           jax.ShapeDtypeStruct((ew, D), jnp.float32))
    k = pl.kernel(
        _sc_gather_body,
        out_type=out,
        mesh=plsc.VectorSubcoreMesh(core_axis_name="c", subcore_axis_name="s",
                                    num_cores=NC, num_subcores=NS),
        scratch_types=[
            pltpu.VMEM((G_CH,), jnp.int32),
            pltpu.VMEM((G_CH, D), jnp.float32),
            pltpu.SemaphoreType.DMA,
        ],
    )
    return k(pam, pbm, paw, pbw, sm, rm, sw, rw)


def _mlp_body(ga_ref, gb_ref, e_ref, w1c_ref, w2_ref, b2_ref, out_ref):
    e = e_ref[...]
    h = ga_ref[...] + gb_ref[...] + jnp.dot(e, w1c_ref[...],
                                            preferred_element_type=jnp.float32)
    h = jnp.maximum(h, 0.0)
    out_ref[...] = e + jnp.dot(h, w2_ref[...],
                               preferred_element_type=jnp.float32) + b2_ref[...]


def _edge_mlp(ga, gb, e, w1c, w2, b2, blk):
    n = e.shape[0]
    row_spec = pl.BlockSpec((blk, D), lambda i: (i, 0))
    wspec = pl.BlockSpec((D, D), lambda i: (0, 0))
    bias = pl.BlockSpec((1, D), lambda i: (0, 0))
    return pl.pallas_call(
        _mlp_body,
        grid=(n // blk,),
        in_specs=[row_spec, row_spec, row_spec, wspec, wspec, bias],
        out_specs=row_spec,
        out_shape=jax.ShapeDtypeStruct((n, D), jnp.float32),
        compiler_params=pltpu.CompilerParams(
            dimension_semantics=("arbitrary",)),
    )(ga, gb, e, w1c, w2, b2.reshape(1, D))


def kernel(x, mesh_edge_index, mesh_edge_attr, world_edge_index, world_edge_attr,
           Wm1, bm1, Wm2, bm2, Ww1, bw1, Ww2, bw2):
    pam, pbm, paw, pbw = _precompute_tables(x, Wm1, bm1, Ww1, bw1)
    gam, gbm, gaw, gbw = _sc_gather(
        pam, pbm, paw, pbw,
        mesh_edge_index[0], mesh_edge_index[1],
        world_edge_index[0], world_edge_index[1])
    mesh_out = _edge_mlp(gam, gbm, mesh_edge_attr, Wm1[2 * D:], Wm2, bm2, 4000)
    world_out = _edge_mlp(gaw, gbw, world_edge_attr, Ww1[2 * D:], Ww2, bw2, 4000)
    return (mesh_out, world_out)
